# Initial kernel scaffold; baseline (speedup 1.0000x reference)
import jax
import jax.numpy as jnp
from jax.experimental import pallas as pl


def kernel(x):
    raise NotImplementedError("write your pallas kernel here")


# speedup vs baseline: 8.9957x; 8.9957x over previous
import functools

import jax
import jax.numpy as jnp
from jax.experimental import pallas as pl

SEQ_LEN = 12
BATCH = 2
N_NODES = 2048
DIM = 64
K = 5
KPAD = 8
ROWS = 256


def _knn_kernel(x_ref, val_ref, ind_ref):
    i = pl.program_id(1)
    X = x_ref[0]
    nrm = jnp.sqrt(jnp.sum(X * X, axis=1, keepdims=True))
    Xn = X / nrm
    Xr = x_ref[0, pl.ds(i * ROWS, ROWS), :]
    rnrm = jnp.sqrt(jnp.sum(Xr * Xr, axis=1, keepdims=True))
    rows = Xr / rnrm
    dist = jax.lax.dot_general(
        rows, Xn,
        dimension_numbers=(((1,), (1,)), ((), ())),
        preferred_element_type=jnp.float32,
    )
    colid = jax.lax.broadcasted_iota(jnp.int32, (ROWS, N_NODES), 1)
    vals = []
    inds = []
    d = dist
    for _ in range(K):
        m = jnp.max(d, axis=1)
        is_m = d == m[:, None]
        am = jnp.min(jnp.where(is_m, colid, N_NODES), axis=1)
        lv = jnp.where(m >= 0, m, 0.01 * m)
        vals.append(lv)
        inds.append(am)
        d = jnp.where(colid == am[:, None], -jnp.inf, d)
    zf = jnp.zeros((ROWS,), jnp.float32)
    zi = jnp.zeros((ROWS,), jnp.int32)
    val8 = jnp.stack(vals + [zf] * (KPAD - K), axis=1)
    ind8 = jnp.stack(inds + [zi] * (KPAD - K), axis=1)
    val_ref[0] = val8
    ind_ref[0] = ind8


def _scatter_kernel(val_ref, ind_ref, out_ref):
    i = pl.program_id(1)
    i0 = i * ROWS
    V = val_ref[0]
    I = ind_ref[0]
    Vr = val_ref[0, pl.ds(i0, ROWS), :]
    Ir = ind_ref[0, pl.ds(i0, ROWS), :]
    col = jax.lax.broadcasted_iota(jnp.int32, (ROWS, N_NODES), 1)
    row = jax.lax.broadcasted_iota(jnp.int32, (ROWS, N_NODES), 0) + i0
    acc = jnp.zeros((ROWS, N_NODES), jnp.float32)
    for k in range(K):
        acc += jnp.where(col == Ir[:, k:k + 1], Vr[:, k:k + 1], 0.0)
        ik = jnp.reshape(I[:, k], (1, N_NODES))
        vk = jnp.reshape(V[:, k], (1, N_NODES))
        acc += jnp.where(row == ik, vk, 0.0)
    out_ref[0] = acc * 0.5


@jax.jit
def kernel(x):
    T, B, N, D = x.shape
    xr = jnp.transpose(x, (1, 2, 0, 3)).reshape(B, N, T * D)
    nb = N // ROWS
    val8, ind8 = pl.pallas_call(
        _knn_kernel,
        grid=(B, nb),
        in_specs=[pl.BlockSpec((1, N, T * D), lambda b, i: (b, 0, 0))],
        out_specs=[
            pl.BlockSpec((1, ROWS, KPAD), lambda b, i: (b, i, 0)),
            pl.BlockSpec((1, ROWS, KPAD), lambda b, i: (b, i, 0)),
        ],
        out_shape=[
            jax.ShapeDtypeStruct((B, N, KPAD), jnp.float32),
            jax.ShapeDtypeStruct((B, N, KPAD), jnp.int32),
        ],
    )(xr)
    out = pl.pallas_call(
        _scatter_kernel,
        grid=(B, nb),
        in_specs=[
            pl.BlockSpec((1, N, KPAD), lambda b, i: (b, 0, 0)),
            pl.BlockSpec((1, N, KPAD), lambda b, i: (b, 0, 0)),
        ],
        out_specs=pl.BlockSpec((1, ROWS, N), lambda b, i: (b, i, 0)),
        out_shape=jax.ShapeDtypeStruct((B, N, N), jnp.float32),
    )(val8, ind8)
    return out


# speedup vs baseline: 11.6613x; 1.2963x over previous
import functools

import jax
import jax.numpy as jnp
from jax.experimental import pallas as pl
from jax.experimental.pallas import tpu as pltpu

SEQ_LEN = 12
BATCH = 2
N_NODES = 2048
DIM = 64
K = 5
KPAD = 8
ROWS = 256


def _knn_kernel(x_ref, vnk_ref, ink_ref, vkn_ref, ikn_ref, xn_ref):
    i = pl.program_id(1)

    @pl.when(i == 0)
    def _normalize():
        X = x_ref[0]
        nrm = jnp.sqrt(jnp.sum(X * X, axis=1, keepdims=True))
        xn_ref[...] = X / nrm

    Xn = xn_ref[...]
    rows = xn_ref[pl.ds(i * ROWS, ROWS), :]
    dist = jax.lax.dot_general(
        rows, Xn,
        dimension_numbers=(((1,), (1,)), ((), ())),
        preferred_element_type=jnp.float32,
    )
    colid = jax.lax.broadcasted_iota(jnp.int32, (ROWS, N_NODES), 1)
    vals = []
    inds = []
    d = dist
    for _ in range(K):
        m = jnp.max(d, axis=1)
        is_m = d == m[:, None]
        am = jnp.min(jnp.where(is_m, colid, N_NODES), axis=1)
        lv = jnp.where(m >= 0, m, 0.01 * m)
        vals.append(lv)
        inds.append(am)
        d = jnp.where(colid == am[:, None], -jnp.inf, d)
    zf = jnp.zeros((ROWS,), jnp.float32)
    zi = jnp.zeros((ROWS,), jnp.int32)
    vnk_ref[0] = jnp.stack(vals + [zf] * (KPAD - K), axis=1)
    ink_ref[0] = jnp.stack(inds + [zi] * (KPAD - K), axis=1)
    vkn_ref[0] = jnp.stack(vals + [zf] * (KPAD - K), axis=0)
    ikn_ref[0] = jnp.stack(inds + [zi] * (KPAD - K), axis=0)


def _scatter_kernel(vnk_ref, ink_ref, vkn_ref, ikn_ref, out_ref):
    i = pl.program_id(1)
    i0 = i * ROWS
    Vr = vnk_ref[0]
    Ir = ink_ref[0]
    Vt = vkn_ref[0]
    It = ikn_ref[0]
    col = jax.lax.broadcasted_iota(jnp.int32, (ROWS, N_NODES), 1)
    row = jax.lax.broadcasted_iota(jnp.int32, (ROWS, N_NODES), 0) + i0
    acc = jnp.zeros((ROWS, N_NODES), jnp.float32)
    for k in range(K):
        acc += jnp.where(col == Ir[:, k:k + 1], Vr[:, k:k + 1], 0.0)
        acc += jnp.where(row == It[k:k + 1, :], Vt[k:k + 1, :], 0.0)
    out_ref[0] = acc * 0.5


@jax.jit
def kernel(x):
    T, B, N, D = x.shape
    xr = jnp.transpose(x, (1, 2, 0, 3)).reshape(B, N, T * D)
    nb = N // ROWS
    vnk, ink, vkn, ikn = pl.pallas_call(
        _knn_kernel,
        grid=(B, nb),
        in_specs=[pl.BlockSpec((1, N, T * D), lambda b, i: (b, 0, 0))],
        out_specs=[
            pl.BlockSpec((1, ROWS, KPAD), lambda b, i: (b, i, 0)),
            pl.BlockSpec((1, ROWS, KPAD), lambda b, i: (b, i, 0)),
            pl.BlockSpec((1, KPAD, ROWS), lambda b, i: (b, 0, i)),
            pl.BlockSpec((1, KPAD, ROWS), lambda b, i: (b, 0, i)),
        ],
        out_shape=[
            jax.ShapeDtypeStruct((B, N, KPAD), jnp.float32),
            jax.ShapeDtypeStruct((B, N, KPAD), jnp.int32),
            jax.ShapeDtypeStruct((B, KPAD, N), jnp.float32),
            jax.ShapeDtypeStruct((B, KPAD, N), jnp.int32),
        ],
        scratch_shapes=[pltpu.VMEM((N, T * D), jnp.float32)],
    )(xr)
    out = pl.pallas_call(
        _scatter_kernel,
        grid=(B, nb),
        in_specs=[
            pl.BlockSpec((1, ROWS, KPAD), lambda b, i: (b, i, 0)),
            pl.BlockSpec((1, ROWS, KPAD), lambda b, i: (b, i, 0)),
            pl.BlockSpec((1, KPAD, N), lambda b, i: (b, 0, 0)),
            pl.BlockSpec((1, KPAD, N), lambda b, i: (b, 0, 0)),
        ],
        out_specs=pl.BlockSpec((1, ROWS, N), lambda b, i: (b, i, 0)),
        out_shape=jax.ShapeDtypeStruct((B, N, N), jnp.float32),
    )(vnk, ink, vkn, ikn)
    return out


# speedup vs baseline: 16.0934x; 1.3801x over previous
import jax
import jax.numpy as jnp
from jax.experimental import pallas as pl
from jax.experimental.pallas import tpu as pltpu

SEQ_LEN = 12
BATCH = 2
N_NODES = 2048
DIM = 64
K = 5
ROWS = 256
TD = SEQ_LEN * DIM


def _assemble_normalize(x_ref, xn_ref):
    for t in range(SEQ_LEN):
        xn_ref[:, t * DIM:(t + 1) * DIM] = x_ref[t, 0]
    X = xn_ref[...]
    nrm = jnp.sqrt(jnp.sum(X * X, axis=1, keepdims=True))
    xn_ref[...] = X / nrm


def _thresh_kernel(x_ref, tr_ref, tc_ref, xn_ref):
    i = pl.program_id(1)

    @pl.when(i == 0)
    def _():
        _assemble_normalize(x_ref, xn_ref)

    rows = xn_ref[pl.ds(i * ROWS, ROWS), :]
    d = jax.lax.dot_general(
        rows, xn_ref[...],
        dimension_numbers=(((1,), (1,)), ((), ())),
        preferred_element_type=jnp.float32,
    )
    for k in range(K):
        m = jnp.max(d, axis=1)
        if k < K - 1:
            d = jnp.where(d == m[:, None], -jnp.inf, d)
    t5 = m
    tr_ref[0] = jnp.reshape(t5, (ROWS, 1))
    tc_ref[0] = jnp.reshape(t5, (1, ROWS))


def _out_kernel(x_ref, tr_ref, tc_ref, out_ref, xn_ref):
    i = pl.program_id(1)

    @pl.when(i == 0)
    def _():
        _assemble_normalize(x_ref, xn_ref)

    rows = xn_ref[pl.ds(i * ROWS, ROWS), :]
    d = jax.lax.dot_general(
        rows, xn_ref[...],
        dimension_numbers=(((1,), (1,)), ((), ())),
        preferred_element_type=jnp.float32,
    )
    t_r = tr_ref[0]
    t_c = tc_ref[0]
    w = jnp.where(d >= t_r, 0.5, 0.0) + jnp.where(d >= t_c, 0.5, 0.0)
    lv = jnp.where(d >= 0, d, 0.01 * d)
    out_ref[0] = lv * w


@jax.jit
def kernel(x):
    T, B, N, D = x.shape
    nb = N // ROWS
    x_spec = pl.BlockSpec((T, 1, N, D), lambda b, i: (0, b, 0, 0))
    tr, tc = pl.pallas_call(
        _thresh_kernel,
        grid=(B, nb),
        in_specs=[x_spec],
        out_specs=[
            pl.BlockSpec((1, ROWS, 1), lambda b, i: (b, i, 0)),
            pl.BlockSpec((1, 1, ROWS), lambda b, i: (b, 0, i)),
        ],
        out_shape=[
            jax.ShapeDtypeStruct((B, N, 1), jnp.float32),
            jax.ShapeDtypeStruct((B, 1, N), jnp.float32),
        ],
        scratch_shapes=[pltpu.VMEM((N, TD), jnp.float32)],
    )(x)
    out = pl.pallas_call(
        _out_kernel,
        grid=(B, nb),
        in_specs=[
            x_spec,
            pl.BlockSpec((1, ROWS, 1), lambda b, i: (b, i, 0)),
            pl.BlockSpec((1, 1, N), lambda b, i: (b, 0, 0)),
        ],
        out_specs=pl.BlockSpec((1, ROWS, N), lambda b, i: (b, i, 0)),
        out_shape=jax.ShapeDtypeStruct((B, N, N), jnp.float32),
        scratch_shapes=[pltpu.VMEM((N, TD), jnp.float32)],
    )(x, tr, tc)
    return out


# speedup vs baseline: 17.4505x; 1.0843x over previous
import jax
import jax.numpy as jnp
from jax.experimental import pallas as pl
from jax.experimental.pallas import tpu as pltpu

SEQ_LEN = 12
BATCH = 2
N_NODES = 2048
DIM = 64
K = 5
ROWS = 512
TD = SEQ_LEN * DIM


def _assemble_normalize(x_ref, xn_ref):
    for t in range(SEQ_LEN):
        xn_ref[:, t * DIM:(t + 1) * DIM] = x_ref[t, 0]
    X = xn_ref[...]
    nrm = jnp.sqrt(jnp.sum(X * X, axis=1, keepdims=True))
    xn_ref[...] = X / nrm


def _thresh_kernel(x_ref, tr_ref, tc_ref, xn_ref):
    i = pl.program_id(1)

    @pl.when(i == 0)
    def _():
        _assemble_normalize(x_ref, xn_ref)

    rows = xn_ref[pl.ds(i * ROWS, ROWS), :]
    d = jax.lax.dot_general(
        rows, xn_ref[...],
        dimension_numbers=(((1,), (1,)), ((), ())),
        preferred_element_type=jnp.float32,
    )
    for k in range(K):
        m = jnp.max(d, axis=1)
        if k < K - 1:
            d = jnp.where(d == m[:, None], -jnp.inf, d)
    t5 = m
    tr_ref[0] = jnp.reshape(t5, (ROWS, 1))
    tc_ref[0] = jnp.reshape(t5, (1, ROWS))


def _out_kernel(x_ref, tr_ref, tc_ref, out_ref, xn_ref):
    i = pl.program_id(1)

    @pl.when(i == 0)
    def _():
        _assemble_normalize(x_ref, xn_ref)

    rows = xn_ref[pl.ds(i * ROWS, ROWS), :]
    d = jax.lax.dot_general(
        rows, xn_ref[...],
        dimension_numbers=(((1,), (1,)), ((), ())),
        preferred_element_type=jnp.float32,
    )
    t_r = tr_ref[0]
    t_c = tc_ref[0]
    w = jnp.where(d >= t_r, 0.5, 0.0) + jnp.where(d >= t_c, 0.5, 0.0)
    lv = jnp.where(d >= 0, d, 0.01 * d)
    out_ref[0] = lv * w


@jax.jit
def kernel(x):
    T, B, N, D = x.shape
    nb = N // ROWS
    x_spec = pl.BlockSpec((T, 1, N, D), lambda b, i: (0, b, 0, 0))
    tr, tc = pl.pallas_call(
        _thresh_kernel,
        grid=(B, nb),
        in_specs=[x_spec],
        out_specs=[
            pl.BlockSpec((1, ROWS, 1), lambda b, i: (b, i, 0)),
            pl.BlockSpec((1, 1, ROWS), lambda b, i: (b, 0, i)),
        ],
        out_shape=[
            jax.ShapeDtypeStruct((B, N, 1), jnp.float32),
            jax.ShapeDtypeStruct((B, 1, N), jnp.float32),
        ],
        scratch_shapes=[pltpu.VMEM((N, TD), jnp.float32)],
    )(x)
    out = pl.pallas_call(
        _out_kernel,
        grid=(B, nb),
        in_specs=[
            x_spec,
            pl.BlockSpec((1, ROWS, 1), lambda b, i: (b, i, 0)),
            pl.BlockSpec((1, 1, N), lambda b, i: (b, 0, 0)),
        ],
        out_specs=pl.BlockSpec((1, ROWS, N), lambda b, i: (b, i, 0)),
        out_shape=jax.ShapeDtypeStruct((B, N, N), jnp.float32),
        scratch_shapes=[pltpu.VMEM((N, TD), jnp.float32)],
    )(x, tr, tc)
    return out


# speedup vs baseline: 18.3828x; 1.0534x over previous
import jax
import jax.numpy as jnp
from jax.experimental import pallas as pl
from jax.experimental.pallas import tpu as pltpu

SEQ_LEN = 12
BATCH = 2
N_NODES = 2048
DIM = 64
K = 5
ROWS = 1024
TD = SEQ_LEN * DIM


def _assemble_normalize(x_ref, xn_ref):
    for t in range(SEQ_LEN):
        xn_ref[:, t * DIM:(t + 1) * DIM] = x_ref[t, 0]
    X = xn_ref[...]
    nrm = jnp.sqrt(jnp.sum(X * X, axis=1, keepdims=True))
    xn_ref[...] = X / nrm


def _thresh_kernel(x_ref, tr_ref, tc_ref, xn_ref):
    i = pl.program_id(1)

    @pl.when(i == 0)
    def _():
        _assemble_normalize(x_ref, xn_ref)

    rows = xn_ref[pl.ds(i * ROWS, ROWS), :]
    d = jax.lax.dot_general(
        rows, xn_ref[...],
        dimension_numbers=(((1,), (1,)), ((), ())),
        preferred_element_type=jnp.float32,
    )
    for k in range(K):
        m = jnp.max(d, axis=1)
        if k < K - 1:
            d = jnp.where(d == m[:, None], -jnp.inf, d)
    t5 = m
    tr_ref[0] = jnp.reshape(t5, (ROWS, 1))
    tc_ref[0] = jnp.reshape(t5, (1, ROWS))


def _out_kernel(x_ref, tr_ref, tc_ref, out_ref, xn_ref):
    i = pl.program_id(1)

    @pl.when(i == 0)
    def _():
        _assemble_normalize(x_ref, xn_ref)

    rows = xn_ref[pl.ds(i * ROWS, ROWS), :]
    d = jax.lax.dot_general(
        rows, xn_ref[...],
        dimension_numbers=(((1,), (1,)), ((), ())),
        preferred_element_type=jnp.float32,
    )
    t_r = tr_ref[0]
    t_c = tc_ref[0]
    w = jnp.where(d >= t_r, 0.5, 0.0) + jnp.where(d >= t_c, 0.5, 0.0)
    lv = jnp.where(d >= 0, d, 0.01 * d)
    out_ref[0] = lv * w


@jax.jit
def kernel(x):
    T, B, N, D = x.shape
    nb = N // ROWS
    x_spec = pl.BlockSpec((T, 1, N, D), lambda b, i: (0, b, 0, 0))
    tr, tc = pl.pallas_call(
        _thresh_kernel,
        grid=(B, nb),
        in_specs=[x_spec],
        out_specs=[
            pl.BlockSpec((1, ROWS, 1), lambda b, i: (b, i, 0)),
            pl.BlockSpec((1, 1, ROWS), lambda b, i: (b, 0, i)),
        ],
        out_shape=[
            jax.ShapeDtypeStruct((B, N, 1), jnp.float32),
            jax.ShapeDtypeStruct((B, 1, N), jnp.float32),
        ],
        scratch_shapes=[pltpu.VMEM((N, TD), jnp.float32)],
    )(x)
    out = pl.pallas_call(
        _out_kernel,
        grid=(B, nb),
        in_specs=[
            x_spec,
            pl.BlockSpec((1, ROWS, 1), lambda b, i: (b, i, 0)),
            pl.BlockSpec((1, 1, N), lambda b, i: (b, 0, 0)),
        ],
        out_specs=pl.BlockSpec((1, ROWS, N), lambda b, i: (b, i, 0)),
        out_shape=jax.ShapeDtypeStruct((B, N, N), jnp.float32),
        scratch_shapes=[pltpu.VMEM((N, TD), jnp.float32)],
    )(x, tr, tc)
    return out


# speedup vs baseline: 21.0374x; 1.1444x over previous
import jax
import jax.numpy as jnp
from jax.experimental import pallas as pl
from jax.experimental.pallas import tpu as pltpu

SEQ_LEN = 12
BATCH = 2
N_NODES = 2048
DIM = 64
K = 5
RB = 512
TD = SEQ_LEN * DIM
NBB = N_NODES // RB


def _assemble_normalize(x_ref, xn_ref):
    for t in range(SEQ_LEN):
        xn_ref[:, t * DIM:(t + 1) * DIM] = x_ref[t, 0]
    X = xn_ref[...]
    nrm = jnp.sqrt(jnp.sum(X * X, axis=1, keepdims=True))
    xn_ref[...] = X / nrm


def _dist_rows(xn_ref, i0, nrows):
    rows = xn_ref[pl.ds(i0, nrows), :]
    return jax.lax.dot_general(
        rows, xn_ref[...],
        dimension_numbers=(((1,), (1,)), ((), ())),
        preferred_element_type=jnp.float32,
    )


def _fused_kernel(x_ref, out_ref, xn_ref, tr_ref, tc_ref):
    p = pl.program_id(1)

    @pl.when(p == 0)
    def _thresholds():
        _assemble_normalize(x_ref, xn_ref)
        for j in range(NBB):
            d = _dist_rows(xn_ref, j * RB, RB)
            for k in range(K):
                m = jnp.max(d, axis=1)
                if k < K - 1:
                    d = jnp.where(d == m[:, None], -jnp.inf, d)
            tr_ref[pl.ds(j * RB, RB), :] = jnp.reshape(m, (RB, 1))
            tc_ref[:, pl.ds(j * RB, RB)] = jnp.reshape(m, (1, RB))

    @pl.when(p > 0)
    def _emit():
        i0 = (p - 1) * RB
        d = _dist_rows(xn_ref, i0, RB)
        t_r = tr_ref[pl.ds(i0, RB), :]
        t_c = tc_ref[...]
        w = jnp.where(d >= t_r, 0.5, 0.0) + jnp.where(d >= t_c, 0.5, 0.0)
        lv = jnp.where(d >= 0, d, 0.01 * d)
        out_ref[0] = lv * w


@jax.jit
def kernel(x):
    T, B, N, D = x.shape
    out = pl.pallas_call(
        _fused_kernel,
        grid=(B, 1 + NBB),
        in_specs=[pl.BlockSpec((T, 1, N, D), lambda b, p: (0, b, 0, 0))],
        out_specs=pl.BlockSpec(
            (1, RB, N), lambda b, p: (b, jnp.maximum(p - 1, 0), 0)),
        out_shape=jax.ShapeDtypeStruct((B, N, N), jnp.float32),
        scratch_shapes=[
            pltpu.VMEM((N, TD), jnp.float32),
            pltpu.VMEM((N, 1), jnp.float32),
            pltpu.VMEM((1, N), jnp.float32),
        ],
    )(x)
    return out


# speedup vs baseline: 21.2470x; 1.0100x over previous
import jax
import jax.numpy as jnp
from jax.experimental import pallas as pl
from jax.experimental.pallas import tpu as pltpu

SEQ_LEN = 12
BATCH = 2
N_NODES = 2048
DIM = 64
K = 5
RB = 256
TD = SEQ_LEN * DIM
NBB = N_NODES // RB


def _assemble_normalize(x_ref, xn_ref):
    for t in range(SEQ_LEN):
        xn_ref[:, t * DIM:(t + 1) * DIM] = x_ref[t, 0]
    X = xn_ref[...]
    nrm = jnp.sqrt(jnp.sum(X * X, axis=1, keepdims=True))
    xn_ref[...] = X / nrm


def _dist_rows(xn_ref, i0, nrows):
    rows = xn_ref[pl.ds(i0, nrows), :]
    return jax.lax.dot_general(
        rows, xn_ref[...],
        dimension_numbers=(((1,), (1,)), ((), ())),
        preferred_element_type=jnp.float32,
    )


def _fused_kernel(x_ref, out_ref, xn_ref, d_ref, tr_ref, tc_ref):
    p = pl.program_id(1)

    @pl.when(p == 0)
    def _thresholds():
        _assemble_normalize(x_ref, xn_ref)
        for j in range(NBB):
            d = _dist_rows(xn_ref, j * RB, RB)
            d_ref[pl.ds(j * RB, RB), :] = d
            for k in range(K):
                m = jnp.max(d, axis=1)
                if k < K - 1:
                    d = jnp.where(d == m[:, None], -jnp.inf, d)
            tr_ref[pl.ds(j * RB, RB), :] = jnp.reshape(m, (RB, 1))
            tc_ref[:, pl.ds(j * RB, RB)] = jnp.reshape(m, (1, RB))

    @pl.when(p > 0)
    def _emit():
        i0 = (p - 1) * RB
        d = d_ref[pl.ds(i0, RB), :]
        t_r = tr_ref[pl.ds(i0, RB), :]
        t_c = tc_ref[...]
        w = jnp.where(d >= t_r, 0.5, 0.0) + jnp.where(d >= t_c, 0.5, 0.0)
        lv = jnp.where(d >= 0, d, 0.01 * d)
        out_ref[0] = lv * w


@jax.jit
def kernel(x):
    T, B, N, D = x.shape
    out = pl.pallas_call(
        _fused_kernel,
        grid=(B, 1 + NBB),
        in_specs=[pl.BlockSpec((T, 1, N, D), lambda b, p: (0, b, 0, 0))],
        out_specs=pl.BlockSpec(
            (1, RB, N), lambda b, p: (b, jnp.maximum(p - 1, 0), 0)),
        out_shape=jax.ShapeDtypeStruct((B, N, N), jnp.float32),
        scratch_shapes=[
            pltpu.VMEM((N, TD), jnp.float32),
            pltpu.VMEM((N, N), jnp.float32),
            pltpu.VMEM((N, 1), jnp.float32),
            pltpu.VMEM((1, N), jnp.float32),
        ],
    )(x)
    return out
